# baseline (speedup 1.0000x reference)
import jax
import jax.numpy as jnp
from jax.experimental import pallas as pl
from jax.experimental.pallas import tpu as pltpu

D_MODEL = 384
N_HEAD = 6
HEAD_DIM = 64
NUM_EXPERTS = 65536
K_ACT = 16
EH = 128
SQRT_N = 256
SUB = 32
S = 512
NBUF = 2
TTOK = 2
NROW = TTOK * N_HEAD * K_ACT
NSTEP = S // TTOK


def _dot_bf16(a, b, dims):
    return jax.lax.dot_general(
        a.astype(jnp.bfloat16), b.astype(jnp.bfloat16), (dims, ((), ())),
        preferred_element_type=jnp.float32)


def _round_bf16(x):
    return x.astype(jnp.bfloat16).astype(jnp.float32)


def _top16(s, n):
    rows = s.shape[0]
    iota_n = jax.lax.broadcasted_iota(jnp.int32, (rows, n), 1)
    iota_k = jax.lax.broadcasted_iota(jnp.int32, (rows, K_ACT), 1)

    def body(i, carry):
        sc, vals, idxs = carry
        m = jnp.max(sc, axis=-1, keepdims=True)
        am = jnp.min(jnp.where(sc == m, iota_n, n), axis=-1, keepdims=True)
        vals = jnp.where(iota_k == i, m, vals)
        idxs = jnp.where(iota_k == i, am, idxs)
        sc = jnp.where(iota_n == am, -jnp.inf, sc)
        return (sc, vals, idxs)

    init = (s, jnp.zeros((rows, K_ACT), jnp.float32), jnp.zeros((rows, K_ACT), jnp.int32))
    _, vals, idxs = jax.lax.fori_loop(0, K_ACT, body, init)
    return vals, idxs


def _routing_kernel(x_ref, wq_ref, ck_ref, cpk_ref, g_ref, b_ref, gi_ref, rw_ref):
    x = x_ref[...]
    wqh = wq_ref[...]
    q = _dot_bf16(x, wqh, ((1,), (1,)))
    mu = jnp.mean(q, axis=-1, keepdims=True)
    qc = q - mu
    var = jnp.mean(qc * qc, axis=-1, keepdims=True)
    qn = qc / jnp.sqrt(var + 1e-5) * g_ref[...] + b_ref[...]
    s1 = _dot_bf16(qn[:, :SUB], ck_ref[...], ((1,), (1,)))
    s2 = _dot_bf16(qn[:, SUB:], cpk_ref[...], ((1,), (1,)))
    v1, i1 = _top16(s1, SQRT_N)
    v2, i2 = _top16(s2, SQRT_N)
    v1e = jnp.concatenate(
        [jnp.broadcast_to(v1[:, a:a + 1], (S, K_ACT)) for a in range(K_ACT)], axis=1)
    v2e = jnp.concatenate([v2] * K_ACT, axis=1)
    joint = v1e + v2e
    fs, fidx = _top16(joint, K_ACT * K_ACT)
    a_rank = fidx // K_ACT
    b_rank = fidx - a_rank * K_ACT
    real_row = jnp.zeros((S, K_ACT), jnp.int32)
    real_col = jnp.zeros((S, K_ACT), jnp.int32)
    for j in range(K_ACT):
        real_row = jnp.where(a_rank == j,
                             jnp.broadcast_to(i1[:, j:j + 1], (S, K_ACT)), real_row)
        real_col = jnp.where(b_rank == j,
                             jnp.broadcast_to(i2[:, j:j + 1], (S, K_ACT)), real_col)
    gi = real_row * SQRT_N + real_col
    e = jnp.exp(fs - jnp.max(fs, axis=-1, keepdims=True))
    rw = e / jnp.sum(e, axis=-1, keepdims=True)
    gi_ref[...] = gi.reshape(1, S, K_ACT)
    rw_ref[...] = rw.reshape(1, S, K_ACT)


def _fold_lanes(v, times):
    for _ in range(times):
        half = v.shape[1] // 2
        v = v[:, :half] + v[:, half:]
    return v


def _fold_rows(v, times):
    for _ in range(times):
        half = v.shape[0] // 2
        v = v[:half, :] + v[half:, :]
    return v


def _ffn_kernel(idx_ref, x_ref, rw_ref, exp_ref, q_ref, r_ref, sum_ref,
                wd_ref, wu_ref, out_ref, dbuf, ubuf, dsem, usem):
    t = pl.program_id(0)

    def issue(tn):
        slot = tn % NBUF
        for tt in range(TTOK):
            for h in range(N_HEAD):
                base = h * S * K_ACT + tn * TTOK * K_ACT + tt * K_ACT
                for k in range(K_ACT):
                    idx = idx_ref[base + k]
                    j = tt * N_HEAD * K_ACT + h * K_ACT + k
                    pltpu.make_async_copy(wd_ref.at[idx], dbuf.at[slot, j], dsem.at[slot]).start()
                    pltpu.make_async_copy(wu_ref.at[idx], ubuf.at[slot, j], usem.at[slot]).start()

    @pl.when(t == 0)
    def _prologue():
        for d in range(NBUF - 1):
            issue(jnp.int32(d))

    slot = t % NBUF
    pltpu.make_async_copy(wd_ref.at[pl.ds(0, NROW)], dbuf.at[slot], dsem.at[slot]).wait()
    pltpu.make_async_copy(wu_ref.at[pl.ds(0, NROW)], ubuf.at[slot], usem.at[slot]).wait()

    tn = t + NBUF - 1

    @pl.when(tn < NSTEP)
    def _steady():
        issue(tn)

    xh12 = x_ref[pl.ds(t * TTOK, TTOK), :, :].reshape(TTOK * N_HEAD, HEAD_DIM)
    w8 = _round_bf16(dbuf[slot])
    u8 = _round_bf16(ubuf[slot])
    bdot = lambda a, bb: jax.lax.dot_general(
        a, bb, ((((1,), (0,))), ((), ())), preferred_element_type=jnp.float32)
    x192 = bdot(exp_ref[...], xh12.astype(jnp.bfloat16))
    xrep = bdot(x192.astype(jnp.bfloat16), q_ref[...])
    hcol = xrep * w8
    hid = _fold_lanes(hcol, 6)
    hid = 0.5 * hid * (1.0 + jax.lax.erf(hid * 0.7071067811865476))
    rwslice = rw_ref[pl.ds(t * TTOK, TTOK), :]
    eye2 = (jax.lax.broadcasted_iota(jnp.int32, (TTOK, TTOK), 0)
            == jax.lax.broadcasted_iota(jnp.int32, (TTOK, TTOK), 1)).astype(jnp.float32)
    rwt = jax.lax.dot_general(rwslice, eye2, ((((0,), (0,))), ((), ())),
                              preferred_element_type=jnp.float32,
                              precision=jax.lax.Precision.HIGHEST)
    rwcol = jnp.concatenate([rwt[:, 0:1], rwt[:, 1:2]], axis=0)
    hw = hid * rwcol
    g8 = bdot(hw.astype(jnp.bfloat16), r_ref[...])
    p8 = g8 * u8
    pf = _fold_lanes(p8, 7)
    out12 = jax.lax.dot_general(sum_ref[...], pf, ((((1,), (0,))), ((), ())),
                                preferred_element_type=jnp.float32,
                                precision=jax.lax.Precision.HIGHEST)
    out12 = _round_bf16(out12)
    out_ref[pl.ds(t * TTOK, TTOK), :, :] = out12.reshape(TTOK, N_HEAD, HEAD_DIM)


def _proj_kernel(oh_ref, wo_ref, o_ref):
    o_ref[...] = _dot_bf16(oh_ref[...], wo_ref[...], ((1,), (1,)))


def kernel(x, Wq, Wo, c_keys, c_prime_keys, ln_g, ln_b, w_down, w_up):
    b, s_len, d = x.shape
    x2 = x.reshape(S, D_MODEL)

    gi, rw = pl.pallas_call(
        _routing_kernel,
        grid=(N_HEAD,),
        in_specs=[
            pl.BlockSpec((S, D_MODEL), lambda h: (0, 0)),
            pl.BlockSpec((HEAD_DIM, D_MODEL), lambda h: (h, 0)),
            pl.BlockSpec((SQRT_N, SUB), lambda h: (0, 0)),
            pl.BlockSpec((SQRT_N, SUB), lambda h: (0, 0)),
            pl.BlockSpec((1, HEAD_DIM), lambda h: (0, 0)),
            pl.BlockSpec((1, HEAD_DIM), lambda h: (0, 0)),
        ],
        out_specs=[
            pl.BlockSpec((1, S, K_ACT), lambda h: (h, 0, 0)),
            pl.BlockSpec((1, S, K_ACT), lambda h: (h, 0, 0)),
        ],
        out_shape=[
            jax.ShapeDtypeStruct((N_HEAD, S, K_ACT), jnp.int32),
            jax.ShapeDtypeStruct((N_HEAD, S, K_ACT), jnp.float32),
        ],
    )(x2, Wq, c_keys, c_prime_keys, ln_g.reshape(1, HEAD_DIM), ln_b.reshape(1, HEAD_DIM))

    idx_flat = gi.reshape(-1)
    rwq = rw.transpose(1, 0, 2).reshape(S, N_HEAD * K_ACT)
    x3 = x.reshape(S, N_HEAD, HEAD_DIM)

    NTH = TTOK * N_HEAD
    expmat = (jax.lax.broadcasted_iota(jnp.int32, (NROW, NTH), 0) // K_ACT
              == jax.lax.broadcasted_iota(jnp.int32, (NROW, NTH), 1)
              ).astype(jnp.bfloat16)
    qmat = (jax.lax.broadcasted_iota(jnp.int32, (HEAD_DIM, HEAD_DIM * EH), 1) // EH
            == jax.lax.broadcasted_iota(jnp.int32, (HEAD_DIM, HEAD_DIM * EH), 0)
            ).astype(jnp.bfloat16)
    rmat = (jax.lax.broadcasted_iota(jnp.int32, (EH, EH * HEAD_DIM), 1) // HEAD_DIM
            == jax.lax.broadcasted_iota(jnp.int32, (EH, EH * HEAD_DIM), 0)
            ).astype(jnp.bfloat16)
    sum12 = (jax.lax.broadcasted_iota(jnp.int32, (NTH, NROW), 1) // K_ACT
             == jax.lax.broadcasted_iota(jnp.int32, (NTH, NROW), 0)
             ).astype(jnp.float32)

    oh = pl.pallas_call(
        _ffn_kernel,
        grid_spec=pltpu.PrefetchScalarGridSpec(
            num_scalar_prefetch=1,
            grid=(NSTEP,),
            in_specs=[
                pl.BlockSpec((S, N_HEAD, HEAD_DIM), lambda t, *_: (0, 0, 0)),
                pl.BlockSpec((S, N_HEAD * K_ACT), lambda t, *_: (0, 0)),
                pl.BlockSpec((NROW, NTH), lambda t, *_: (0, 0)),
                pl.BlockSpec((HEAD_DIM, HEAD_DIM * EH), lambda t, *_: (0, 0)),
                pl.BlockSpec((EH, EH * HEAD_DIM), lambda t, *_: (0, 0)),
                pl.BlockSpec((NTH, NROW), lambda t, *_: (0, 0)),
                pl.BlockSpec(memory_space=pl.ANY),
                pl.BlockSpec(memory_space=pl.ANY),
            ],
            out_specs=pl.BlockSpec((S, N_HEAD, HEAD_DIM), lambda t, *_: (0, 0, 0)),
            scratch_shapes=[
                pltpu.VMEM((NBUF, NROW, HEAD_DIM * EH), jnp.float32),
                pltpu.VMEM((NBUF, NROW, EH * HEAD_DIM), jnp.float32),
                pltpu.SemaphoreType.DMA((NBUF,)),
                pltpu.SemaphoreType.DMA((NBUF,)),
            ],
        ),
        out_shape=jax.ShapeDtypeStruct((S, N_HEAD, HEAD_DIM), jnp.float32),
    )(idx_flat, x3, rwq, expmat, qmat, rmat, sum12, w_down, w_up)

    out = pl.pallas_call(
        _proj_kernel,
        in_specs=[
            pl.BlockSpec((S, D_MODEL), lambda: (0, 0)),
            pl.BlockSpec((D_MODEL, D_MODEL), lambda: (0, 0)),
        ],
        out_specs=pl.BlockSpec((S, D_MODEL), lambda: (0, 0)),
        out_shape=jax.ShapeDtypeStruct((S, D_MODEL), jnp.float32),
    )(oh.reshape(S, D_MODEL), Wo)

    return (out.reshape(b, s_len, d), jnp.float32(0.0))


# speedup vs baseline: 1.0983x; 1.0983x over previous
import jax
import jax.numpy as jnp
from jax.experimental import pallas as pl
from jax.experimental.pallas import tpu as pltpu

D_MODEL = 384
N_HEAD = 6
HEAD_DIM = 64
NUM_EXPERTS = 65536
K_ACT = 16
EH = 128
SQRT_N = 256
SUB = 32
S = 512
NBUF = 3
TTOK = 2
NROW = TTOK * N_HEAD * K_ACT
NSTEP = S // TTOK


def _dot_bf16(a, b, dims):
    return jax.lax.dot_general(
        a.astype(jnp.bfloat16), b.astype(jnp.bfloat16), (dims, ((), ())),
        preferred_element_type=jnp.float32)


def _round_bf16(x):
    return x.astype(jnp.bfloat16).astype(jnp.float32)


def _top16(s, n):
    rows = s.shape[0]
    iota_n = jax.lax.broadcasted_iota(jnp.int32, (rows, n), 1)
    iota_k = jax.lax.broadcasted_iota(jnp.int32, (rows, K_ACT), 1)

    def body(i, carry):
        sc, vals, idxs = carry
        m = jnp.max(sc, axis=-1, keepdims=True)
        am = jnp.min(jnp.where(sc == m, iota_n, n), axis=-1, keepdims=True)
        vals = jnp.where(iota_k == i, m, vals)
        idxs = jnp.where(iota_k == i, am, idxs)
        sc = jnp.where(iota_n == am, -jnp.inf, sc)
        return (sc, vals, idxs)

    init = (s, jnp.zeros((rows, K_ACT), jnp.float32), jnp.zeros((rows, K_ACT), jnp.int32))
    _, vals, idxs = jax.lax.fori_loop(0, K_ACT, body, init)
    return vals, idxs


def _routing_kernel(x_ref, wq_ref, ck_ref, cpk_ref, g_ref, b_ref, gi_ref, rw_ref):
    x = x_ref[...]
    wqh = wq_ref[...]
    q = _dot_bf16(x, wqh, ((1,), (1,)))
    mu = jnp.mean(q, axis=-1, keepdims=True)
    qc = q - mu
    var = jnp.mean(qc * qc, axis=-1, keepdims=True)
    qn = qc / jnp.sqrt(var + 1e-5) * g_ref[...] + b_ref[...]
    s1 = _dot_bf16(qn[:, :SUB], ck_ref[...], ((1,), (1,)))
    s2 = _dot_bf16(qn[:, SUB:], cpk_ref[...], ((1,), (1,)))
    v1, i1 = _top16(s1, SQRT_N)
    v2, i2 = _top16(s2, SQRT_N)
    v1e = jnp.concatenate(
        [jnp.broadcast_to(v1[:, a:a + 1], (S, K_ACT)) for a in range(K_ACT)], axis=1)
    v2e = jnp.concatenate([v2] * K_ACT, axis=1)
    joint = v1e + v2e
    fs, fidx = _top16(joint, K_ACT * K_ACT)
    a_rank = fidx // K_ACT
    b_rank = fidx - a_rank * K_ACT
    real_row = jnp.zeros((S, K_ACT), jnp.int32)
    real_col = jnp.zeros((S, K_ACT), jnp.int32)
    for j in range(K_ACT):
        real_row = jnp.where(a_rank == j,
                             jnp.broadcast_to(i1[:, j:j + 1], (S, K_ACT)), real_row)
        real_col = jnp.where(b_rank == j,
                             jnp.broadcast_to(i2[:, j:j + 1], (S, K_ACT)), real_col)
    gi = real_row * SQRT_N + real_col
    e = jnp.exp(fs - jnp.max(fs, axis=-1, keepdims=True))
    rw = e / jnp.sum(e, axis=-1, keepdims=True)
    gi_ref[...] = gi.reshape(1, S, K_ACT)
    rw_ref[...] = rw.reshape(1, S, K_ACT)


def _fold_lanes(v, times):
    for _ in range(times):
        half = v.shape[1] // 2
        v = v[:, :half] + v[:, half:]
    return v


def _fold_rows(v, times):
    for _ in range(times):
        half = v.shape[0] // 2
        v = v[:half, :] + v[half:, :]
    return v


def _ffn_kernel(idx_ref, x_ref, rw_ref, exp_ref, q_ref, r_ref, sum_ref,
                wd_ref, wu_ref, out_ref, dbuf, ubuf, dsem, usem):
    t = pl.program_id(0)

    def issue(tn):
        slot = tn % NBUF
        for tt in range(TTOK):
            for h in range(N_HEAD):
                base = h * S * K_ACT + tn * TTOK * K_ACT + tt * K_ACT
                for k in range(K_ACT):
                    idx = idx_ref[base + k]
                    j = tt * N_HEAD * K_ACT + h * K_ACT + k
                    pltpu.make_async_copy(wd_ref.at[idx], dbuf.at[slot, j], dsem.at[slot]).start()
                    pltpu.make_async_copy(wu_ref.at[idx], ubuf.at[slot, j], usem.at[slot]).start()

    @pl.when(t == 0)
    def _prologue():
        for d in range(NBUF - 1):
            issue(jnp.int32(d))

    slot = t % NBUF
    pltpu.make_async_copy(wd_ref.at[pl.ds(0, NROW)], dbuf.at[slot], dsem.at[slot]).wait()
    pltpu.make_async_copy(wu_ref.at[pl.ds(0, NROW)], ubuf.at[slot], usem.at[slot]).wait()

    tn = t + NBUF - 1

    @pl.when(tn < NSTEP)
    def _steady():
        issue(tn)

    xh12 = x_ref[pl.ds(t * TTOK, TTOK), :, :].reshape(TTOK * N_HEAD, HEAD_DIM)
    w8 = _round_bf16(dbuf[slot])
    u8 = _round_bf16(ubuf[slot])
    bdot = lambda a, bb: jax.lax.dot_general(
        a, bb, ((((1,), (0,))), ((), ())), preferred_element_type=jnp.float32)
    x192 = bdot(exp_ref[...], xh12.astype(jnp.bfloat16))
    xrep = bdot(x192.astype(jnp.bfloat16), q_ref[...])
    hcol = xrep * w8
    hid = _fold_lanes(hcol, 6)
    hid = 0.5 * hid * (1.0 + jax.lax.erf(hid * 0.7071067811865476))
    rwslice = rw_ref[pl.ds(t * TTOK, TTOK), :]
    eye2 = (jax.lax.broadcasted_iota(jnp.int32, (TTOK, TTOK), 0)
            == jax.lax.broadcasted_iota(jnp.int32, (TTOK, TTOK), 1)).astype(jnp.float32)
    rwt = jax.lax.dot_general(rwslice, eye2, ((((0,), (0,))), ((), ())),
                              preferred_element_type=jnp.float32,
                              precision=jax.lax.Precision.HIGHEST)
    rwcol = jnp.concatenate([rwt[:, 0:1], rwt[:, 1:2]], axis=0)
    hw = hid * rwcol
    g8 = bdot(hw.astype(jnp.bfloat16), r_ref[...])
    p8 = g8 * u8
    pf = _fold_lanes(p8, 7)
    out12 = jax.lax.dot_general(sum_ref[...], pf, ((((1,), (0,))), ((), ())),
                                preferred_element_type=jnp.float32,
                                precision=jax.lax.Precision.HIGHEST)
    out12 = _round_bf16(out12)
    out_ref[pl.ds(t * TTOK, TTOK), :, :] = out12.reshape(TTOK, N_HEAD, HEAD_DIM)


def _proj_kernel(oh_ref, wo_ref, o_ref):
    o_ref[...] = _dot_bf16(oh_ref[...], wo_ref[...], ((1,), (1,)))


def kernel(x, Wq, Wo, c_keys, c_prime_keys, ln_g, ln_b, w_down, w_up):
    b, s_len, d = x.shape
    x2 = x.reshape(S, D_MODEL)

    gi, rw = pl.pallas_call(
        _routing_kernel,
        grid=(N_HEAD,),
        in_specs=[
            pl.BlockSpec((S, D_MODEL), lambda h: (0, 0)),
            pl.BlockSpec((HEAD_DIM, D_MODEL), lambda h: (h, 0)),
            pl.BlockSpec((SQRT_N, SUB), lambda h: (0, 0)),
            pl.BlockSpec((SQRT_N, SUB), lambda h: (0, 0)),
            pl.BlockSpec((1, HEAD_DIM), lambda h: (0, 0)),
            pl.BlockSpec((1, HEAD_DIM), lambda h: (0, 0)),
        ],
        out_specs=[
            pl.BlockSpec((1, S, K_ACT), lambda h: (h, 0, 0)),
            pl.BlockSpec((1, S, K_ACT), lambda h: (h, 0, 0)),
        ],
        out_shape=[
            jax.ShapeDtypeStruct((N_HEAD, S, K_ACT), jnp.int32),
            jax.ShapeDtypeStruct((N_HEAD, S, K_ACT), jnp.float32),
        ],
    )(x2, Wq, c_keys, c_prime_keys, ln_g.reshape(1, HEAD_DIM), ln_b.reshape(1, HEAD_DIM))

    idx_flat = gi.reshape(-1)
    rwq = rw.transpose(1, 0, 2).reshape(S, N_HEAD * K_ACT)
    x3 = x.reshape(S, N_HEAD, HEAD_DIM)

    NTH = TTOK * N_HEAD
    expmat = (jax.lax.broadcasted_iota(jnp.int32, (NROW, NTH), 0) // K_ACT
              == jax.lax.broadcasted_iota(jnp.int32, (NROW, NTH), 1)
              ).astype(jnp.bfloat16)
    qmat = (jax.lax.broadcasted_iota(jnp.int32, (HEAD_DIM, HEAD_DIM * EH), 1) // EH
            == jax.lax.broadcasted_iota(jnp.int32, (HEAD_DIM, HEAD_DIM * EH), 0)
            ).astype(jnp.bfloat16)
    rmat = (jax.lax.broadcasted_iota(jnp.int32, (EH, EH * HEAD_DIM), 1) // HEAD_DIM
            == jax.lax.broadcasted_iota(jnp.int32, (EH, EH * HEAD_DIM), 0)
            ).astype(jnp.bfloat16)
    sum12 = (jax.lax.broadcasted_iota(jnp.int32, (NTH, NROW), 1) // K_ACT
             == jax.lax.broadcasted_iota(jnp.int32, (NTH, NROW), 0)
             ).astype(jnp.float32)

    oh = pl.pallas_call(
        _ffn_kernel,
        grid_spec=pltpu.PrefetchScalarGridSpec(
            num_scalar_prefetch=1,
            grid=(NSTEP,),
            in_specs=[
                pl.BlockSpec((S, N_HEAD, HEAD_DIM), lambda t, *_: (0, 0, 0)),
                pl.BlockSpec((S, N_HEAD * K_ACT), lambda t, *_: (0, 0)),
                pl.BlockSpec((NROW, NTH), lambda t, *_: (0, 0)),
                pl.BlockSpec((HEAD_DIM, HEAD_DIM * EH), lambda t, *_: (0, 0)),
                pl.BlockSpec((EH, EH * HEAD_DIM), lambda t, *_: (0, 0)),
                pl.BlockSpec((NTH, NROW), lambda t, *_: (0, 0)),
                pl.BlockSpec(memory_space=pl.ANY),
                pl.BlockSpec(memory_space=pl.ANY),
            ],
            out_specs=pl.BlockSpec((S, N_HEAD, HEAD_DIM), lambda t, *_: (0, 0, 0)),
            scratch_shapes=[
                pltpu.VMEM((NBUF, NROW, HEAD_DIM * EH), jnp.float32),
                pltpu.VMEM((NBUF, NROW, EH * HEAD_DIM), jnp.float32),
                pltpu.SemaphoreType.DMA((NBUF,)),
                pltpu.SemaphoreType.DMA((NBUF,)),
            ],
        ),
        out_shape=jax.ShapeDtypeStruct((S, N_HEAD, HEAD_DIM), jnp.float32),
    )(idx_flat, x3, rwq, expmat, qmat, rmat, sum12, w_down, w_up)

    out = pl.pallas_call(
        _proj_kernel,
        in_specs=[
            pl.BlockSpec((S, D_MODEL), lambda: (0, 0)),
            pl.BlockSpec((D_MODEL, D_MODEL), lambda: (0, 0)),
        ],
        out_specs=pl.BlockSpec((S, D_MODEL), lambda: (0, 0)),
        out_shape=jax.ShapeDtypeStruct((S, D_MODEL), jnp.float32),
    )(oh.reshape(S, D_MODEL), Wo)

    return (out.reshape(b, s_len, d), jnp.float32(0.0))
